# Initial kernel scaffold; baseline (speedup 1.0000x reference)
import jax
import jax.numpy as jnp
from jax.experimental import pallas as pl


def kernel(x, w_conv, w_tab, bias_node):
    raise NotImplementedError("write your pallas kernel here")


# speedup vs baseline: 1.6013x; 1.6013x over previous
import jax
import jax.numpy as jnp
from jax.experimental import pallas as pl
from jax.experimental.pallas import tpu as pltpu

_TID_ROWS = 288
_DIW_ROWS = 7


def _st_kernel(xt_ref, w1t_ref, w2t_ref, biast_ref, o_ref):
    f = xt_ref[0]
    n = f.shape[1]
    t = w2t_ref.shape[1]

    tid = jnp.clip((f[34, :] * 288.0).astype(jnp.int32), 0, _TID_ROWS - 1)
    diw = jnp.clip(f[35, :].astype(jnp.int32), 0, _DIW_ROWS - 1) + _TID_ROWS

    row = jax.lax.broadcasted_iota(jnp.int32, (t, n), 0)
    onehot = jnp.logical_or(row == tid[None, :],
                            row == diw[None, :]).astype(jnp.float32)

    acc = jnp.dot(w1t_ref[...], f, preferred_element_type=jnp.float32)
    acc = acc + jnp.dot(w2t_ref[...], onehot, preferred_element_type=jnp.float32)
    o_ref[0] = acc + biast_ref[...]


def kernel(x, w_conv, w_tab, bias_node):
    B, L, N, C = x.shape
    K, Eo = w_conv.shape
    T = w_tab.shape[0]

    xt = jnp.transpose(x, (0, 1, 3, 2)).reshape(B, K, N)
    w1t = w_conv.T
    w2t = w_tab.T
    biast = bias_node.T

    out = pl.pallas_call(
        _st_kernel,
        out_shape=jax.ShapeDtypeStruct((B, Eo, N), jnp.float32),
        grid=(B,),
        in_specs=[
            pl.BlockSpec((1, K, N), lambda i: (i, 0, 0)),
            pl.BlockSpec((Eo, K), lambda i: (0, 0)),
            pl.BlockSpec((Eo, T), lambda i: (0, 0)),
            pl.BlockSpec((Eo, N), lambda i: (0, 0)),
        ],
        out_specs=pl.BlockSpec((1, Eo, N), lambda i: (i, 0, 0)),
        compiler_params=pltpu.CompilerParams(dimension_semantics=("parallel",)),
    )(xt, w1t, w2t, biast)

    return out[..., None]


# speedup vs baseline: 2.9855x; 1.8644x over previous
import jax
import jax.numpy as jnp
from jax.experimental import pallas as pl
from jax.experimental.pallas import tpu as pltpu

_TID = 288
_DIW = 7
_BB = 8


def _st_kernel(xt_ref, idx_ref, w1t_ref, wtt_ref, wdt_ref, bt_ref, o_ref):
    n = xt_ref.shape[2]
    row_t = jax.lax.broadcasted_iota(jnp.int32, (_TID, n), 0)
    row_d = jax.lax.broadcasted_iota(jnp.int32, (8, n), 0)
    bias = bt_ref[...]
    for j in range(_BB):
        f = xt_ref[j]
        tid = idx_ref[j, 0]
        diw = idx_ref[j, 1]
        oh_t = (row_t == tid[None, :]).astype(jnp.bfloat16)
        oh_d = (row_d == diw[None, :]).astype(jnp.bfloat16)
        acc = jnp.dot(w1t_ref[...], f, preferred_element_type=jnp.float32)
        acc = acc + jnp.dot(wtt_ref[...], oh_t, preferred_element_type=jnp.float32)
        acc = acc + jnp.dot(wdt_ref[...], oh_d, preferred_element_type=jnp.float32)
        o_ref[j] = acc + bias


def kernel(x, w_conv, w_tab, bias_node):
    B, L, N, C = x.shape
    K, Eo = w_conv.shape

    xt = jnp.transpose(x, (0, 1, 3, 2)).reshape(B, K, N).astype(jnp.bfloat16)
    tid = jnp.clip((x[:, -1, :, 1] * 288.0).astype(jnp.int32), 0, _TID - 1)
    diw = jnp.clip(x[:, -1, :, 2].astype(jnp.int32), 0, _DIW - 1)
    idx = jnp.stack([tid, diw], axis=1)

    w1t = w_conv.T.astype(jnp.bfloat16)
    wtt = w_tab[:_TID].T.astype(jnp.bfloat16)
    wdt = w_tab[_TID:_TID + 8].T.astype(jnp.bfloat16)
    biast = bias_node.T

    out = pl.pallas_call(
        _st_kernel,
        out_shape=jax.ShapeDtypeStruct((B, Eo, N), jnp.float32),
        grid=(B // _BB,),
        in_specs=[
            pl.BlockSpec((_BB, K, N), lambda i: (i, 0, 0)),
            pl.BlockSpec((_BB, 2, N), lambda i: (i, 0, 0)),
            pl.BlockSpec((Eo, K), lambda i: (0, 0)),
            pl.BlockSpec((Eo, _TID), lambda i: (0, 0)),
            pl.BlockSpec((Eo, 8), lambda i: (0, 0)),
            pl.BlockSpec((Eo, N), lambda i: (0, 0)),
        ],
        out_specs=pl.BlockSpec((_BB, Eo, N), lambda i: (i, 0, 0)),
        compiler_params=pltpu.CompilerParams(dimension_semantics=("parallel",)),
    )(xt, idx, w1t, wtt, wdt, biast)

    return out[..., None]
